# baseline (speedup 1.0000x reference)
import jax
import jax.numpy as jnp
from jax.experimental import pallas as pl
from jax.experimental.pallas import tpu as pltpu

_NUM_CODES = 1024
_DIM = 256


def _rowsum256_lanes(sq):
    t = sq[:, :128] + sq[:, 128:]
    acc = t[:, 0:8]
    for k in range(1, 16):
        acc = acc + t[:, 8 * k:8 * k + 8]
    u = acc[:, 0:4] + acc[:, 4:8]
    w = u[:, 0:2] + u[:, 2:4]
    return w[:, 0:1] + w[:, 1:2]


def _rowsum256_sublanes(sq):
    t = sq[:128, :] + sq[128:, :]
    acc = t[0:8, :]
    for k in range(1, 16):
        acc = acc + t[8 * k:8 * k + 8, :]
    u = acc[0:4, :] + acc[4:8, :]
    w = u[0:2, :] + u[2:4, :]
    return w[0:1, :] + w[1:2, :]


def _vq_body(z_ref, emb_ref, q_ref, vq_ref, cm_ref, acc_ref):
    i = pl.program_id(0)
    zn = z_ref[0]
    emb = emb_ref[...]
    c = jax.lax.dot_general(
        emb, zn, (((1,), (0,)), ((), ())),
        preferred_element_type=jnp.float32)
    a2 = _rowsum256_sublanes(zn * zn)
    b2 = _rowsum256_lanes(emb * emb)
    d2 = a2 + b2 - 2.0 * c
    d = jnp.sqrt(jnp.maximum(d2, 0.0))
    m = jnp.min(d, axis=0, keepdims=True)
    iota = jax.lax.broadcasted_iota(jnp.int32, d.shape, 0)
    idx = jnp.min(jnp.where(d == m, iota, _NUM_CODES), axis=0, keepdims=True)
    onehot = (iota == idx).astype(jnp.float32)
    q = jax.lax.dot_general(
        emb, onehot, (((0,), (0,)), ((), ())),
        precision=jax.lax.Precision.HIGHEST,
        preferred_element_type=jnp.float32)
    q_ref[0] = q
    diff = zn - q
    part = jnp.sum(diff * diff)

    @pl.when(i == 0)
    def _init():
        acc_ref[0] = 0.0

    acc_ref[0] += part

    @pl.when(i == pl.num_programs(0) - 1)
    def _finish():
        vq = acc_ref[0] / jnp.float32(16 * 1024 * _DIM)
        vq_ref[0, 0] = vq
        cm_ref[0, 0] = 0.25 * vq


def kernel(z, embedding):
    n, chan, h, w = z.shape
    zf = z.reshape(n, chan, h * w)
    q, vq, cm = pl.pallas_call(
        _vq_body,
        grid=(n,),
        in_specs=[
            pl.BlockSpec((1, chan, h * w), lambda i: (i, 0, 0)),
            pl.BlockSpec((_NUM_CODES, _DIM), lambda i: (0, 0)),
        ],
        out_specs=[
            pl.BlockSpec((1, chan, h * w), lambda i: (i, 0, 0)),
            pl.BlockSpec((1, 1), lambda i: (0, 0), memory_space=pltpu.SMEM),
            pl.BlockSpec((1, 1), lambda i: (0, 0), memory_space=pltpu.SMEM),
        ],
        out_shape=[
            jax.ShapeDtypeStruct((n, chan, h * w), jnp.float32),
            jax.ShapeDtypeStruct((1, 1), jnp.float32),
            jax.ShapeDtypeStruct((1, 1), jnp.float32),
        ],
        scratch_shapes=[pltpu.SMEM((1,), jnp.float32)],
    )(zf, embedding)
    return q.reshape(n, chan, h, w), vq[0, 0], cm[0, 0]


# speedup vs baseline: 1.1920x; 1.1920x over previous
import jax
import jax.numpy as jnp
from jax.experimental import pallas as pl
from jax.experimental.pallas import tpu as pltpu

_NUM_CODES = 1024
_DIM = 256


def _rowsum256_lanes(sq):
    t = sq[:, :128] + sq[:, 128:]
    acc = t[:, 0:8]
    for k in range(1, 16):
        acc = acc + t[:, 8 * k:8 * k + 8]
    u = acc[:, 0:4] + acc[:, 4:8]
    w = u[:, 0:2] + u[:, 2:4]
    return w[:, 0:1] + w[:, 1:2]


def _rowsum256_sublanes(sq):
    t = sq[:128, :] + sq[128:, :]
    acc = t[0:8, :]
    for k in range(1, 16):
        acc = acc + t[8 * k:8 * k + 8, :]
    u = acc[0:4, :] + acc[4:8, :]
    w = u[0:2, :] + u[2:4, :]
    return w[0:1, :] + w[1:2, :]


def _vq_body(z_ref, emb_ref, q_ref, vq_ref, cm_ref, acc_ref, b2_ref):
    i = pl.program_id(0)
    zn = z_ref[0]
    emb = emb_ref[...]

    @pl.when(i == 0)
    def _compute_b2():
        b2_ref[...] = _rowsum256_lanes(emb * emb)

    c = jax.lax.dot_general(
        emb, zn, (((1,), (0,)), ((), ())),
        preferred_element_type=jnp.float32)
    a2 = _rowsum256_sublanes(zn * zn)
    b2 = b2_ref[...]
    d2 = a2 + b2 - 2.0 * c
    d = jnp.sqrt(jnp.maximum(d2, 0.0))
    m = jnp.min(d, axis=0, keepdims=True)
    iota = jax.lax.broadcasted_iota(jnp.int32, d.shape, 0)
    idx = jnp.min(jnp.where(d == m, iota, _NUM_CODES), axis=0, keepdims=True)
    onehot = (iota == idx).astype(jnp.float32)
    q = jax.lax.dot_general(
        emb, onehot, (((0,), (0,)), ((), ())),
        precision=jax.lax.Precision.HIGHEST,
        preferred_element_type=jnp.float32)
    q_ref[0] = q
    diff = zn - q
    part = jnp.sum(diff * diff)

    @pl.when(i == 0)
    def _init():
        acc_ref[0] = 0.0

    acc_ref[0] += part

    @pl.when(i == pl.num_programs(0) - 1)
    def _finish():
        vq = acc_ref[0] / jnp.float32(16 * 1024 * _DIM)
        vq_ref[0, 0] = vq
        cm_ref[0, 0] = 0.25 * vq


def kernel(z, embedding):
    n, chan, h, w = z.shape
    zf = z.reshape(n, chan, h * w)
    q, vq, cm = pl.pallas_call(
        _vq_body,
        grid=(n,),
        in_specs=[
            pl.BlockSpec((1, chan, h * w), lambda i: (i, 0, 0)),
            pl.BlockSpec((_NUM_CODES, _DIM), lambda i: (0, 0)),
        ],
        out_specs=[
            pl.BlockSpec((1, chan, h * w), lambda i: (i, 0, 0)),
            pl.BlockSpec((1, 1), lambda i: (0, 0), memory_space=pltpu.SMEM),
            pl.BlockSpec((1, 1), lambda i: (0, 0), memory_space=pltpu.SMEM),
        ],
        out_shape=[
            jax.ShapeDtypeStruct((n, chan, h * w), jnp.float32),
            jax.ShapeDtypeStruct((1, 1), jnp.float32),
            jax.ShapeDtypeStruct((1, 1), jnp.float32),
        ],
        scratch_shapes=[pltpu.SMEM((1,), jnp.float32),
                        pltpu.VMEM((_NUM_CODES, 1), jnp.float32)],
    )(zf, embedding)
    return q.reshape(n, chan, h, w), vq[0, 0], cm[0, 0]


# speedup vs baseline: 1.1972x; 1.0043x over previous
import jax
import jax.numpy as jnp
from jax.experimental import pallas as pl
from jax.experimental.pallas import tpu as pltpu

_NUM_CODES = 1024
_DIM = 256
_N_IMG = 16
_PX = 1024


def _rowsum256_lanes(sq):
    t = sq[:, :128] + sq[:, 128:]
    acc = t[:, 0:8]
    for k in range(1, 16):
        acc = acc + t[:, 8 * k:8 * k + 8]
    u = acc[:, 0:4] + acc[:, 4:8]
    w = u[:, 0:2] + u[:, 2:4]
    return w[:, 0:1] + w[:, 1:2]


def _rowsum256_sublanes(sq):
    t = sq[:128, :] + sq[128:, :]
    acc = t[0:8, :]
    for k in range(1, 16):
        acc = acc + t[8 * k:8 * k + 8, :]
    u = acc[0:4, :] + acc[4:8, :]
    w = u[0:2, :] + u[2:4, :]
    return w[0:1, :] + w[1:2, :]


def _vq_body(z_ref, emb_ref, q_ref, vq_ref, cm_ref,
             acc_ref, b2_ref, c_buf, a2_buf):
    i = pl.program_id(0)
    emb = emb_ref[...]

    @pl.when(i == 0)
    def _compute_b2():
        b2_ref[...] = _rowsum256_lanes(emb * emb)
        acc_ref[0] = 0.0

    cur = jax.lax.rem(i, 2)
    prv = 1 - cur

    @pl.when(i < _N_IMG)
    def _produce():
        zn = z_ref[0]
        c_buf[cur] = jax.lax.dot_general(
            emb, zn, (((1,), (0,)), ((), ())),
            preferred_element_type=jnp.float32)
        a2_buf[pl.ds(cur, 1), :] = _rowsum256_sublanes(zn * zn)

    @pl.when(i > 0)
    def _consume():
        c = c_buf[prv]
        a2 = a2_buf[pl.ds(prv, 1), :]
        d2 = a2 + b2_ref[...] - 2.0 * c
        d = jnp.sqrt(jnp.maximum(d2, 0.0))
        m = jnp.min(d, axis=0, keepdims=True)
        iota = jax.lax.broadcasted_iota(jnp.int32, d.shape, 0)
        idx = jnp.min(jnp.where(d == m, iota, _NUM_CODES), axis=0,
                      keepdims=True)
        onehot = (iota == idx).astype(jnp.float32)
        q_ref[0] = jax.lax.dot_general(
            emb, onehot, (((0,), (0,)), ((), ())),
            precision=jax.lax.Precision.HIGHEST,
            preferred_element_type=jnp.float32)
        acc_ref[0] += jnp.sum(m * m)

    @pl.when(i == _N_IMG)
    def _finish():
        vq = acc_ref[0] / jnp.float32(_N_IMG * _PX * _DIM)
        vq_ref[0, 0] = vq
        cm_ref[0, 0] = 0.25 * vq


def kernel(z, embedding):
    n, chan, h, w = z.shape
    zf = z.reshape(n, chan, h * w)
    q, vq, cm = pl.pallas_call(
        _vq_body,
        grid=(n + 1,),
        in_specs=[
            pl.BlockSpec((1, chan, h * w),
                         lambda i: (jnp.minimum(i, _N_IMG - 1), 0, 0)),
            pl.BlockSpec((_NUM_CODES, _DIM), lambda i: (0, 0)),
        ],
        out_specs=[
            pl.BlockSpec((1, chan, h * w),
                         lambda i: (jnp.maximum(i - 1, 0), 0, 0)),
            pl.BlockSpec((1, 1), lambda i: (0, 0), memory_space=pltpu.SMEM),
            pl.BlockSpec((1, 1), lambda i: (0, 0), memory_space=pltpu.SMEM),
        ],
        out_shape=[
            jax.ShapeDtypeStruct((n, chan, h * w), jnp.float32),
            jax.ShapeDtypeStruct((1, 1), jnp.float32),
            jax.ShapeDtypeStruct((1, 1), jnp.float32),
        ],
        scratch_shapes=[
            pltpu.SMEM((1,), jnp.float32),
            pltpu.VMEM((_NUM_CODES, 1), jnp.float32),
            pltpu.VMEM((2, _NUM_CODES, _PX), jnp.float32),
            pltpu.VMEM((2, _PX), jnp.float32),
        ],
    )(zf, embedding)
    return q.reshape(n, chan, h, w), vq[0, 0], cm[0, 0]
